# Initial kernel scaffold; baseline (speedup 1.0000x reference)
import jax
import jax.numpy as jnp
from jax.experimental import pallas as pl


def kernel(x, edge_index, edge_weight, batch, embed, W_ggc, W_ih, W_hh, b_ih, b_hh, W1_w, W1_b, W2_w, W2_b, q_w, q_b, W3_w, W3_b):
    raise NotImplementedError("write your pallas kernel here")


# speedup vs baseline: 3.4305x; 3.4305x over previous
import functools

import jax
import jax.numpy as jnp
from jax import lax
from jax.experimental import pallas as pl
from jax.experimental.pallas import tpu as pltpu
from jax.experimental.pallas import tpu_sc as plsc

N = 10000
E = 320000
NV = 100000
H = 128
B = 256

NC = 2
NS = 16
NW = NC * NS
NP = 10240
GPW = NP // NW
EPW = 10112
EP = NW * EPW
CK = 128
NCHUNK = EPW // CK

RB = 512
VB = 4096

@functools.cache
def _build_sc_kernels():
    mesh = plsc.VectorSubcoreMesh(core_axis_name="c", subcore_axis_name="s",
                                  num_cores=NC, num_subcores=NS)

    @functools.partial(
        pl.kernel, mesh=mesh,
        out_type=jax.ShapeDtypeStruct((NP, H), jnp.float32),
        scratch_types=[
            pltpu.VMEM((4, 80), jnp.int32),
            pltpu.VMEM((GPW, H), jnp.float32),
            pltpu.SemaphoreType.DMA,
        ],
    )
    def gather_k(x_hbm, embed_hbm, out_hbm, idx_v, rows_v, sem):
        wid = lax.axis_index("s") * NC + lax.axis_index("c")
        base = wid * GPW
        for i in range(4):
            pltpu.sync_copy(x_hbm.at[pl.ds(base + i * 80, 80)], idx_v.at[i])
        for i in range(4):
            pltpu.async_copy(embed_hbm.at[idx_v.at[i]],
                             rows_v.at[pl.ds(i * 80, 80)], sem).wait()
        pltpu.sync_copy(rows_v, out_hbm.at[pl.ds(base, GPW)])

    @functools.partial(
        pl.kernel, mesh=mesh,
        compiler_params=pltpu.CompilerParams(needs_layout_passes=False),
        out_type=jax.ShapeDtypeStruct((NC * NP, H), jnp.float32),
        scratch_types=[
            pltpu.VMEM((CK,), jnp.int32),
            pltpu.VMEM((CK,), jnp.int32),
            pltpu.VMEM((CK,), jnp.float32),
            pltpu.VMEM((CK, H), jnp.float32),
            pltpu.VMEM_SHARED((NP, H), jnp.float32),
            pltpu.SemaphoreType.DMA,
        ],
    )
    def edges_k(src_hbm, dst_hbm, w_hbm, m_hbm, out_hbm,
                src_v, dst_v, w_v, rows_v, agg_s, sem):
        cid = lax.axis_index("c")
        sid = lax.axis_index("s")
        wid = sid * NC + cid

        zero16 = jnp.zeros((16,), jnp.float32)

        def _zrow(r, carry):
            for j in range(H // 16):
                rows_v[r, pl.ds(j * 16, 16)] = zero16
            return carry

        lax.fori_loop(0, CK, _zrow, 0)
        rows_per_sub = NP // NS
        for i in range(rows_per_sub // CK):
            pltpu.sync_copy(rows_v,
                            agg_s.at[pl.ds(sid * rows_per_sub + i * CK, CK)])
        plsc.subcore_barrier()

        base = wid * EPW

        def _chunk(c, carry):
            off = base + c * CK
            pltpu.sync_copy(src_hbm.at[pl.ds(off, CK)], src_v)
            pltpu.sync_copy(dst_hbm.at[pl.ds(off, CK)], dst_v)
            pltpu.sync_copy(w_hbm.at[pl.ds(off, CK)], w_v)
            pltpu.async_copy(m_hbm.at[src_v], rows_v, sem).wait()

            def _scale(r, carry2):
                wk = plsc.load_gather(w_v, [jnp.full((16,), r, jnp.int32)])
                for j in range(H // 16):
                    rows_v[r, pl.ds(j * 16, 16)] = (
                        rows_v[r, pl.ds(j * 16, 16)] * wk)
                return carry2

            lax.fori_loop(0, CK, _scale, 0)
            pltpu.sync_copy(rows_v, agg_s.at[dst_v], add=True)
            return carry

        lax.fori_loop(0, NCHUNK, _chunk, 0)
        plsc.subcore_barrier()

        for i in range(rows_per_sub // CK):
            r0 = sid * rows_per_sub + i * CK
            pltpu.sync_copy(agg_s.at[pl.ds(r0, CK)], rows_v)
            pltpu.sync_copy(rows_v, out_hbm.at[pl.ds(cid * NP + r0, CK)])

    return gather_k, edges_k


def _sc_gather(xp, embed):
    return _build_sc_kernels()[0](xp, embed)


def _sc_edges(src_p, dst_p, w_p, m):
    return _build_sc_kernels()[1](src_p, dst_p, w_p, m)


def _mm_body(h_ref, w_ref, o_ref):
    o_ref[...] = jnp.dot(h_ref[...], w_ref[...],
                         preferred_element_type=jnp.float32)


def _gru_body(h_ref, a0_ref, a1_ref, b3_ref, wih_ref, whh_ref, bih_ref,
              bhh_ref, v_ref, last_ref):
    i = pl.program_id(0)
    agg = a0_ref[...] + a1_ref[...]
    h = h_ref[...]
    gi = jax.lax.dot_general(agg, wih_ref[...], (((1,), (1,)), ((), ())),
                             preferred_element_type=jnp.float32) + bih_ref[...]
    gh = jax.lax.dot_general(h, whh_ref[...], (((1,), (1,)), ((), ())),
                             preferred_element_type=jnp.float32) + bhh_ref[...]
    r = jax.nn.sigmoid(gi[:, :H] + gh[:, :H])
    z = jax.nn.sigmoid(gi[:, H:2 * H] + gh[:, H:2 * H])
    n = jnp.tanh(gi[:, 2 * H:] + r * gh[:, 2 * H:])
    v_ref[...] = (1.0 - z) * n + z * h

    bb = b3_ref[0, 0, :]
    gid = i * RB + lax.broadcasted_iota(jnp.int32, (B, RB), 1)
    eq = bb[None, :] == lax.broadcasted_iota(jnp.int32, (B, RB), 0)
    cand = jnp.where(eq, gid, -1)
    bmax = jnp.max(cand, axis=1)[None, :]

    @pl.when(i == 0)
    def _():
        last_ref[...] = jnp.full((1, B), -1, jnp.int32)

    last_ref[...] = jnp.maximum(last_ref[...], bmax)


def _sl_body(v_ref, last_ref, o_ref):
    i = pl.program_id(0)
    lastv = jnp.maximum(last_ref[0, :], 0)
    gid = i * RB + lax.broadcasted_iota(jnp.int32, (B, RB), 1)
    oh = (lastv[:, None] == gid).astype(jnp.float32)
    part = jnp.dot(oh, v_ref[...], preferred_element_type=jnp.float32)

    @pl.when(i == 0)
    def _():
        o_ref[...] = jnp.zeros_like(o_ref)

    o_ref[...] += part


def _sg_body(v_ref, b3_ref, sl_ref, w1_ref, w2_ref, b12_ref, q_ref, qb_ref,
             o_ref):
    i = pl.program_id(0)
    bb = b3_ref[0, 0, :]
    v = v_ref[...]
    oh = (bb[:, None] == lax.broadcasted_iota(jnp.int32, (RB, B), 1)
          ).astype(jnp.float32)
    v_n = jnp.dot(oh, sl_ref[...], preferred_element_type=jnp.float32)
    pre = (jax.lax.dot_general(v_n, w1_ref[...], (((1,), (1,)), ((), ())),
                               preferred_element_type=jnp.float32)
           + jax.lax.dot_general(v, w2_ref[...], (((1,), (1,)), ((), ())),
                                 preferred_element_type=jnp.float32)
           + b12_ref[...])
    sig = jax.nn.sigmoid(pre)
    alpha = jnp.dot(sig, q_ref[...],
                    preferred_element_type=jnp.float32) + qb_ref[0, 0]
    contrib = alpha * v
    part = jax.lax.dot_general(oh, contrib, (((0,), (0,)), ((), ())),
                               preferred_element_type=jnp.float32)

    @pl.when(i == 0)
    def _():
        o_ref[...] = jnp.zeros_like(o_ref)

    o_ref[...] += part


def _sh_body(sl_ref, sg_ref, w3_ref, b3w_ref, o_ref):
    o_ref[...] = (
        jax.lax.dot_general(sl_ref[...], w3_ref[:, :H],
                            (((1,), (1,)), ((), ())),
                            preferred_element_type=jnp.float32)
        + jax.lax.dot_general(sg_ref[...], w3_ref[:, H:],
                              (((1,), (1,)), ((), ())),
                              preferred_element_type=jnp.float32)
        + b3w_ref[...])


def _z_body(sh_ref, emb_ref, o_ref):
    o_ref[...] = jax.lax.dot_general(sh_ref[...], emb_ref[...],
                                     (((1,), (1,)), ((), ())),
                                     preferred_element_type=jnp.float32)


def kernel(x, edge_index, edge_weight, batch, embed, W_ggc, W_ih, W_hh, b_ih,
           b_hh, W1_w, W1_b, W2_w, W2_b, q_w, q_b, W3_w, W3_b):
    xp = jnp.concatenate([x, jnp.zeros((NP - N,), x.dtype)])
    pad_e = EP - E
    pad_idx = (jnp.arange(pad_e, dtype=edge_index.dtype) * 97) % N
    src_p = jnp.concatenate([edge_index[0], pad_idx]).astype(jnp.int32)
    dst_p = jnp.concatenate([edge_index[1], pad_idx]).astype(jnp.int32)
    w_p = jnp.concatenate([edge_weight, jnp.zeros((pad_e,), jnp.float32)])
    batch_p = jnp.concatenate(
        [batch.astype(jnp.int32), jnp.full((NP - N,), 2**30, jnp.int32)])
    batch3 = batch_p.reshape(NP // RB, 1, RB)
    bih2 = b_ih.reshape(1, 3 * H)
    bhh2 = b_hh.reshape(1, 3 * H)
    b12 = (W1_b + W2_b).reshape(1, H)
    qb2 = q_b.reshape(1, 1)
    qmat = jnp.broadcast_to(q_w.reshape(H, 1), (H, H))
    b3w = W3_b.reshape(1, H)

    h = _sc_gather(xp, embed)

    nb = NP // RB
    m = pl.pallas_call(
        _mm_body,
        grid=(nb,),
        in_specs=[pl.BlockSpec((RB, H), lambda i: (i, 0)),
                  pl.BlockSpec((H, H), lambda i: (0, 0))],
        out_specs=pl.BlockSpec((RB, H), lambda i: (i, 0)),
        out_shape=jax.ShapeDtypeStruct((NP, H), jnp.float32),
    )(h, W_ggc)

    agg2 = _sc_edges(src_p, dst_p, w_p, m)

    v, last = pl.pallas_call(
        _gru_body,
        grid=(nb,),
        in_specs=[
            pl.BlockSpec((RB, H), lambda i: (i, 0)),
            pl.BlockSpec((RB, H), lambda i: (i, 0)),
            pl.BlockSpec((RB, H), lambda i: (i + nb, 0)),
            pl.BlockSpec((1, 1, RB), lambda i: (i, 0, 0)),
            pl.BlockSpec((3 * H, H), lambda i: (0, 0)),
            pl.BlockSpec((3 * H, H), lambda i: (0, 0)),
            pl.BlockSpec((1, 3 * H), lambda i: (0, 0)),
            pl.BlockSpec((1, 3 * H), lambda i: (0, 0)),
        ],
        out_specs=[pl.BlockSpec((RB, H), lambda i: (i, 0)),
                   pl.BlockSpec((1, B), lambda i: (0, 0))],
        out_shape=[jax.ShapeDtypeStruct((NP, H), jnp.float32),
                   jax.ShapeDtypeStruct((1, B), jnp.int32)],
    )(h, agg2, agg2, batch3, W_ih, W_hh, bih2, bhh2)

    s_l = pl.pallas_call(
        _sl_body,
        grid=(nb,),
        in_specs=[pl.BlockSpec((RB, H), lambda i: (i, 0)),
                  pl.BlockSpec((1, B), lambda i: (0, 0))],
        out_specs=pl.BlockSpec((B, H), lambda i: (0, 0)),
        out_shape=jax.ShapeDtypeStruct((B, H), jnp.float32),
    )(v, last)

    s_g = pl.pallas_call(
        _sg_body,
        grid=(nb,),
        in_specs=[
            pl.BlockSpec((RB, H), lambda i: (i, 0)),
            pl.BlockSpec((1, 1, RB), lambda i: (i, 0, 0)),
            pl.BlockSpec((B, H), lambda i: (0, 0)),
            pl.BlockSpec((H, H), lambda i: (0, 0)),
            pl.BlockSpec((H, H), lambda i: (0, 0)),
            pl.BlockSpec((1, H), lambda i: (0, 0)),
            pl.BlockSpec((H, H), lambda i: (0, 0)),
            pl.BlockSpec(memory_space=pltpu.SMEM),
        ],
        out_specs=pl.BlockSpec((B, H), lambda i: (0, 0)),
        out_shape=jax.ShapeDtypeStruct((B, H), jnp.float32),
    )(v, batch3, s_l, W1_w, W2_w, b12, qmat, qb2)

    s_h = pl.pallas_call(
        _sh_body,
        in_specs=[pl.BlockSpec((B, H), lambda: (0, 0)),
                  pl.BlockSpec((B, H), lambda: (0, 0)),
                  pl.BlockSpec((H, 2 * H), lambda: (0, 0)),
                  pl.BlockSpec((1, H), lambda: (0, 0))],
        out_specs=pl.BlockSpec((B, H), lambda: (0, 0)),
        out_shape=jax.ShapeDtypeStruct((B, H), jnp.float32),
    )(s_l, s_g, W3_w, b3w)

    nvb = -(-NV // VB)
    z = pl.pallas_call(
        _z_body,
        grid=(nvb,),
        in_specs=[pl.BlockSpec((B, H), lambda i: (0, 0)),
                  pl.BlockSpec((VB, H), lambda i: (i, 0))],
        out_specs=pl.BlockSpec((B, VB), lambda i: (0, i)),
        out_shape=jax.ShapeDtypeStruct((B, NV), jnp.float32),
    )(s_h, embed)

    return z


# speedup vs baseline: 4.8690x; 1.4193x over previous
import functools

import jax
import jax.numpy as jnp
from jax import lax
from jax.experimental import pallas as pl
from jax.experimental.pallas import tpu as pltpu
from jax.experimental.pallas import tpu_sc as plsc

N = 10000
E = 320000
NV = 100000
H = 128
B = 256

NC = 2
NS = 16
NW = NC * NS
NP = 10240
GPW = NP // NW
CK = 128
NCHUNK = 80
CKP = NCHUNK + 2

RB = 512
VB = 4096

@functools.cache
def _build_sc_kernels():
    mesh = plsc.VectorSubcoreMesh(core_axis_name="c", subcore_axis_name="s",
                                  num_cores=NC, num_subcores=NS)

    @functools.partial(
        pl.kernel, mesh=mesh,
        out_type=jax.ShapeDtypeStruct((NP, H), jnp.float32),
        scratch_types=[
            pltpu.VMEM((4, 80), jnp.int32),
            pltpu.VMEM((GPW, H), jnp.float32),
            pltpu.SemaphoreType.DMA,
        ],
    )
    def gather_k(x3_hbm, embed_hbm, out_hbm, idx_v, rows_v, sem):
        wid = lax.axis_index("s") * NC + lax.axis_index("c")
        pltpu.sync_copy(x3_hbm.at[wid], idx_v)
        descs = [
            pltpu.async_copy(embed_hbm.at[idx_v.at[i]],
                             rows_v.at[pl.ds(i * 80, 80)], sem)
            for i in range(4)
        ]
        for d in descs:
            d.wait()
        pltpu.sync_copy(rows_v, out_hbm.at[pl.ds(wid * GPW, GPW)])

    @functools.partial(
        pl.kernel, mesh=mesh,
        compiler_params=pltpu.CompilerParams(needs_layout_passes=False),
        out_type=jax.ShapeDtypeStruct((NC * NP, H), jnp.float32),
        scratch_types=[
            pltpu.VMEM((3, CK), jnp.int32),
            pltpu.VMEM((3, CK), jnp.int32),
            pltpu.VMEM((CK, H), jnp.float32),
            pltpu.VMEM((CK, H), jnp.float32),
            pltpu.VMEM_SHARED((NP, H), jnp.float32),
            pltpu.SemaphoreType.DMA,
            pltpu.SemaphoreType.DMA,
            pltpu.SemaphoreType.DMA,
            pltpu.SemaphoreType.DMA,
        ],
    )
    def edges_k(edata_hbm, m_hbm, out_hbm,
                ebuf_a, ebuf_b, rows_a, rows_b, agg_s,
                gsem_a, gsem_b, ssem_a, ssem_b):
        cid = lax.axis_index("c")
        sid = lax.axis_index("s")
        wid = sid * NC + cid
        ebufs = (ebuf_a, ebuf_b)
        rows = (rows_a, rows_b)
        gsems = (gsem_a, gsem_b)
        ssems = (ssem_a, ssem_b)

        zero16 = jnp.zeros((16,), jnp.float32)

        def _zrow(r, carry):
            for j in range(H // 16):
                rows_a[r, pl.ds(j * 16, 16)] = zero16
            return carry

        lax.fori_loop(0, CK, _zrow, 0)
        rows_per_sub = NP // NS
        for i in range(rows_per_sub // CK):
            pltpu.sync_copy(rows_a,
                            agg_s.at[pl.ds(sid * rows_per_sub + i * CK, CK)])
        plsc.subcore_barrier()

        base = wid * CKP

        def _wait_gather(p):
            pltpu.make_async_copy(m_hbm.at[pl.ds(0, CK)], rows[p],
                                  gsems[p]).wait()

        def _wait_scatter(p):
            pltpu.make_async_copy(rows[p], agg_s.at[pl.ds(0, CK)],
                                  ssems[p]).wait()

        def _fetch(p, blk):
            pltpu.sync_copy(edata_hbm.at[blk], ebufs[p])
            pltpu.async_copy(m_hbm.at[ebufs[p].at[0]], rows[p], gsems[p])

        def _scale(p):
            rv = rows[p]
            eb = ebufs[p]
            two = jnp.full((16,), 2, jnp.int32)

            def _srow(r, carry):
                wk = plsc.bitcast(
                    plsc.load_gather(eb, [two, jnp.full((16,), r, jnp.int32)]),
                    jnp.float32)
                for j in range(H // 16):
                    rv[r, pl.ds(j * 16, 16)] = rv[r, pl.ds(j * 16, 16)] * wk
                return carry

            lax.fori_loop(0, CK, _srow, 0)

        _fetch(0, base)
        _fetch(1, base + 1)

        def _iter(i, carry):
            c = 2 * i
            for p in range(2):
                _wait_gather(p)
                _scale(p)
                pltpu.async_copy(rows[p], agg_s.at[ebufs[p].at[1]], ssems[p],
                                 add=True)
            for p in range(2):
                _wait_scatter(p)
                _fetch(p, base + c + 2 + p)
            return carry

        lax.fori_loop(0, NCHUNK // 2, _iter, 0)
        _wait_gather(0)
        _wait_gather(1)
        plsc.subcore_barrier()

        descs = []
        for i in range(rows_per_sub // CK):
            p = i % 2
            r0 = sid * rows_per_sub + i * CK
            if i >= 2:
                descs[i - 2].wait()
            pltpu.sync_copy(agg_s.at[pl.ds(r0, CK)], rows[p])
            descs.append(
                pltpu.async_copy(rows[p], out_hbm.at[pl.ds(cid * NP + r0, CK)],
                                 gsems[p]))
        for d in descs[-2:]:
            d.wait()

    return gather_k, edges_k


def _sc_gather(x3, embed):
    return _build_sc_kernels()[0](x3, embed)


def _sc_edges(edata, m):
    return _build_sc_kernels()[1](edata, m)


def _mm_body(h_ref, w_ref, o_ref):
    o_ref[...] = jnp.dot(h_ref[...], w_ref[...],
                         preferred_element_type=jnp.float32)


def _gru_body(h_ref, a0_ref, a1_ref, b3_ref, wih_ref, whh_ref, bih_ref,
              bhh_ref, v_ref, last_ref):
    i = pl.program_id(0)
    agg = a0_ref[...] + a1_ref[...]
    h = h_ref[...]
    gi = jax.lax.dot_general(agg, wih_ref[...], (((1,), (1,)), ((), ())),
                             preferred_element_type=jnp.float32) + bih_ref[...]
    gh = jax.lax.dot_general(h, whh_ref[...], (((1,), (1,)), ((), ())),
                             preferred_element_type=jnp.float32) + bhh_ref[...]
    r = jax.nn.sigmoid(gi[:, :H] + gh[:, :H])
    z = jax.nn.sigmoid(gi[:, H:2 * H] + gh[:, H:2 * H])
    n = jnp.tanh(gi[:, 2 * H:] + r * gh[:, 2 * H:])
    v_ref[...] = (1.0 - z) * n + z * h

    bb = b3_ref[0, 0, :]
    gid = i * RB + lax.broadcasted_iota(jnp.int32, (B, RB), 1)
    eq = bb[None, :] == lax.broadcasted_iota(jnp.int32, (B, RB), 0)
    cand = jnp.where(eq, gid, -1)
    bmax = jnp.max(cand, axis=1)[None, :]

    @pl.when(i == 0)
    def _():
        last_ref[...] = jnp.full((1, B), -1, jnp.int32)

    last_ref[...] = jnp.maximum(last_ref[...], bmax)


def _sl_body(v_ref, last_ref, o_ref):
    i = pl.program_id(0)
    lastv = jnp.maximum(last_ref[0, :], 0)
    gid = i * RB + lax.broadcasted_iota(jnp.int32, (B, RB), 1)
    oh = (lastv[:, None] == gid).astype(jnp.float32)
    part = jnp.dot(oh, v_ref[...], preferred_element_type=jnp.float32)

    @pl.when(i == 0)
    def _():
        o_ref[...] = jnp.zeros_like(o_ref)

    o_ref[...] += part


def _sg_body(v_ref, b3_ref, sl_ref, w1_ref, w2_ref, b12_ref, q_ref, qb_ref,
             o_ref):
    i = pl.program_id(0)
    bb = b3_ref[0, 0, :]
    v = v_ref[...]
    oh = (bb[:, None] == lax.broadcasted_iota(jnp.int32, (RB, B), 1)
          ).astype(jnp.float32)
    v_n = jnp.dot(oh, sl_ref[...], preferred_element_type=jnp.float32)
    pre = (jax.lax.dot_general(v_n, w1_ref[...], (((1,), (1,)), ((), ())),
                               preferred_element_type=jnp.float32)
           + jax.lax.dot_general(v, w2_ref[...], (((1,), (1,)), ((), ())),
                                 preferred_element_type=jnp.float32)
           + b12_ref[...])
    sig = jax.nn.sigmoid(pre)
    alpha = jnp.dot(sig, q_ref[...],
                    preferred_element_type=jnp.float32) + qb_ref[0, 0]
    contrib = alpha * v
    part = jax.lax.dot_general(oh, contrib, (((0,), (0,)), ((), ())),
                               preferred_element_type=jnp.float32)

    @pl.when(i == 0)
    def _():
        o_ref[...] = jnp.zeros_like(o_ref)

    o_ref[...] += part


def _sh_body(sl_ref, sg_ref, w3_ref, b3w_ref, o_ref):
    o_ref[...] = (
        jax.lax.dot_general(sl_ref[...], w3_ref[:, :H],
                            (((1,), (1,)), ((), ())),
                            preferred_element_type=jnp.float32)
        + jax.lax.dot_general(sg_ref[...], w3_ref[:, H:],
                              (((1,), (1,)), ((), ())),
                              preferred_element_type=jnp.float32)
        + b3w_ref[...])


def _z_body(sh_ref, emb_ref, o_ref):
    o_ref[...] = jax.lax.dot_general(sh_ref[...], emb_ref[...],
                                     (((1,), (1,)), ((), ())),
                                     preferred_element_type=jnp.float32)


def kernel(x, edge_index, edge_weight, batch, embed, W_ggc, W_ih, W_hh, b_ih,
           b_hh, W1_w, W1_b, W2_w, W2_b, q_w, q_b, W3_w, W3_b):
    xp = jnp.concatenate([x, jnp.zeros((NP - N,), x.dtype)])
    x3 = xp.astype(jnp.int32).reshape(NW, 4, 80)
    pe = NW * NCHUNK * CK
    pad_e = pe - E
    pad_idx = ((jnp.arange(pad_e, dtype=jnp.int32) * 97) % N)
    src_p = jnp.concatenate(
        [edge_index[0].astype(jnp.int32), pad_idx]).reshape(NW, NCHUNK, CK)
    dst_p = jnp.concatenate(
        [edge_index[1].astype(jnp.int32), pad_idx]).reshape(NW, NCHUNK, CK)
    w_p = jnp.concatenate([edge_weight, jnp.zeros((pad_e,), jnp.float32)])
    wbits = jax.lax.bitcast_convert_type(w_p, jnp.int32).reshape(NW, NCHUNK, CK)
    edata = jnp.stack([src_p, dst_p, wbits], axis=2)
    pc_idx = ((jnp.arange(2 * CK, dtype=jnp.int32) * 131) % N).reshape(2, CK)
    pc = jnp.stack([pc_idx, pc_idx, jnp.zeros((2, CK), jnp.int32)], axis=1)
    pc = jnp.broadcast_to(pc[None], (NW, 2, 3, CK))
    edata = jnp.concatenate([edata, pc], axis=1).reshape(NW * CKP, 3, CK)
    batch_p = jnp.concatenate(
        [batch.astype(jnp.int32), jnp.full((NP - N,), 2**30, jnp.int32)])
    batch3 = batch_p.reshape(NP // RB, 1, RB)
    bih2 = b_ih.reshape(1, 3 * H)
    bhh2 = b_hh.reshape(1, 3 * H)
    b12 = (W1_b + W2_b).reshape(1, H)
    qb2 = q_b.reshape(1, 1)
    qmat = jnp.broadcast_to(q_w.reshape(H, 1), (H, H))
    b3w = W3_b.reshape(1, H)

    h = _sc_gather(x3, embed)

    nb = NP // RB
    m = pl.pallas_call(
        _mm_body,
        grid=(nb,),
        in_specs=[pl.BlockSpec((RB, H), lambda i: (i, 0)),
                  pl.BlockSpec((H, H), lambda i: (0, 0))],
        out_specs=pl.BlockSpec((RB, H), lambda i: (i, 0)),
        out_shape=jax.ShapeDtypeStruct((NP, H), jnp.float32),
    )(h, W_ggc)

    agg2 = _sc_edges(edata, m)

    v, last = pl.pallas_call(
        _gru_body,
        grid=(nb,),
        in_specs=[
            pl.BlockSpec((RB, H), lambda i: (i, 0)),
            pl.BlockSpec((RB, H), lambda i: (i, 0)),
            pl.BlockSpec((RB, H), lambda i: (i + nb, 0)),
            pl.BlockSpec((1, 1, RB), lambda i: (i, 0, 0)),
            pl.BlockSpec((3 * H, H), lambda i: (0, 0)),
            pl.BlockSpec((3 * H, H), lambda i: (0, 0)),
            pl.BlockSpec((1, 3 * H), lambda i: (0, 0)),
            pl.BlockSpec((1, 3 * H), lambda i: (0, 0)),
        ],
        out_specs=[pl.BlockSpec((RB, H), lambda i: (i, 0)),
                   pl.BlockSpec((1, B), lambda i: (0, 0))],
        out_shape=[jax.ShapeDtypeStruct((NP, H), jnp.float32),
                   jax.ShapeDtypeStruct((1, B), jnp.int32)],
    )(h, agg2, agg2, batch3, W_ih, W_hh, bih2, bhh2)

    s_l = pl.pallas_call(
        _sl_body,
        grid=(nb,),
        in_specs=[pl.BlockSpec((RB, H), lambda i: (i, 0)),
                  pl.BlockSpec((1, B), lambda i: (0, 0))],
        out_specs=pl.BlockSpec((B, H), lambda i: (0, 0)),
        out_shape=jax.ShapeDtypeStruct((B, H), jnp.float32),
    )(v, last)

    s_g = pl.pallas_call(
        _sg_body,
        grid=(nb,),
        in_specs=[
            pl.BlockSpec((RB, H), lambda i: (i, 0)),
            pl.BlockSpec((1, 1, RB), lambda i: (i, 0, 0)),
            pl.BlockSpec((B, H), lambda i: (0, 0)),
            pl.BlockSpec((H, H), lambda i: (0, 0)),
            pl.BlockSpec((H, H), lambda i: (0, 0)),
            pl.BlockSpec((1, H), lambda i: (0, 0)),
            pl.BlockSpec((H, H), lambda i: (0, 0)),
            pl.BlockSpec(memory_space=pltpu.SMEM),
        ],
        out_specs=pl.BlockSpec((B, H), lambda i: (0, 0)),
        out_shape=jax.ShapeDtypeStruct((B, H), jnp.float32),
    )(v, batch3, s_l, W1_w, W2_w, b12, qmat, qb2)

    s_h = pl.pallas_call(
        _sh_body,
        in_specs=[pl.BlockSpec((B, H), lambda: (0, 0)),
                  pl.BlockSpec((B, H), lambda: (0, 0)),
                  pl.BlockSpec((H, 2 * H), lambda: (0, 0)),
                  pl.BlockSpec((1, H), lambda: (0, 0))],
        out_specs=pl.BlockSpec((B, H), lambda: (0, 0)),
        out_shape=jax.ShapeDtypeStruct((B, H), jnp.float32),
    )(s_l, s_g, W3_w, b3w)

    nvb = -(-NV // VB)
    z = pl.pallas_call(
        _z_body,
        grid=(nvb,),
        in_specs=[pl.BlockSpec((B, H), lambda i: (0, 0)),
                  pl.BlockSpec((VB, H), lambda i: (i, 0))],
        out_specs=pl.BlockSpec((B, VB), lambda i: (0, i)),
        out_shape=jax.ShapeDtypeStruct((B, NV), jnp.float32),
    )(s_h, embed)

    return z


# speedup vs baseline: 6.5723x; 1.3498x over previous
import functools

import jax
import jax.numpy as jnp
from jax import lax
from jax.experimental import pallas as pl
from jax.experimental.pallas import tpu as pltpu
from jax.experimental.pallas import tpu_sc as plsc

N = 10000
E = 320000
NV = 100000
H = 128
B = 256

NC = 2
NS = 16
NW = NC * NS
NP = 10240
GPW = NP // NW
CK = 128
NCHUNK = 80
NBUF = 4
CKP = NCHUNK + NBUF

RB = 512
VB = 4096

@functools.cache
def _build_sc_kernels():
    mesh = plsc.VectorSubcoreMesh(core_axis_name="c", subcore_axis_name="s",
                                  num_cores=NC, num_subcores=NS)

    @functools.partial(
        pl.kernel, mesh=mesh,
        out_type=jax.ShapeDtypeStruct((NP, H), jnp.float32),
        scratch_types=[
            pltpu.VMEM((4, 80), jnp.int32),
            pltpu.VMEM((GPW, H), jnp.float32),
            pltpu.SemaphoreType.DMA,
        ],
    )
    def gather_k(x3_hbm, embed_hbm, out_hbm, idx_v, rows_v, sem):
        wid = lax.axis_index("s") * NC + lax.axis_index("c")
        pltpu.sync_copy(x3_hbm.at[wid], idx_v)
        descs = [
            pltpu.async_copy(embed_hbm.at[idx_v.at[i]],
                             rows_v.at[pl.ds(i * 80, 80)], sem)
            for i in range(4)
        ]
        for d in descs:
            d.wait()
        pltpu.sync_copy(rows_v, out_hbm.at[pl.ds(wid * GPW, GPW)])

    @functools.partial(
        pl.kernel, mesh=mesh,
        compiler_params=pltpu.CompilerParams(needs_layout_passes=False),
        out_type=jax.ShapeDtypeStruct((NC * NP, H), jnp.float32),
        scratch_types=(
            [pltpu.VMEM((4, 2, CK), jnp.int32),
             pltpu.VMEM((CKP * CK,), jnp.float32),
             pltpu.VMEM((2, CK), jnp.int32)]
            + [pltpu.VMEM((CK, H), jnp.float32)] * 2
            + [pltpu.VMEM_SHARED((NP, H), jnp.float32)]
            + [pltpu.SemaphoreType.DMA] * 8
        ),
    )
    def edges_k(edata_hbm, wdata_hbm, m_hbm, out_hbm,
                ebuf, wdata_v, dstb, rows0, rows1, agg_s,
                gs0, gs1, ss0, ss1, es0, es1, es2, es3):
        rows = (rows0, rows1)
        gsems = (gs0, gs1)
        ssems = (ss0, ss1)
        esems = (es0, es1, es2, es3)
        cid = lax.axis_index("c")
        sid = lax.axis_index("s")
        wid = sid * NC + cid

        def _fill_slot(slot, c):
            pltpu.async_copy(edata_hbm.at[wid, c], ebuf.at[slot], esems[slot])

        def _wait_slot(slot):
            pltpu.make_async_copy(edata_hbm.at[wid, 0], ebuf.at[slot],
                                  esems[slot]).wait()

        def _gather(rp, slot):
            pltpu.async_copy(m_hbm.at[ebuf.at[slot, 0]], rows[rp], gsems[rp])

        def _wait_gather(rp):
            pltpu.make_async_copy(m_hbm.at[pl.ds(0, CK)], rows[rp],
                                  gsems[rp]).wait()

        def _wait_scatter(rp):
            pltpu.make_async_copy(rows[rp], agg_s.at[pl.ds(0, CK)],
                                  ssems[rp]).wait()

        def _scale(rp, c):
            rv = rows[rp]
            cbase = c * CK

            def _srow(r2, carry):
                for u in range(2):
                    r = 2 * r2 + u
                    wk = plsc.load_gather(
                        wdata_v, [jnp.full((16,), cbase + r, jnp.int32)])
                    for j in range(H // 16):
                        rv[r, pl.ds(j * 16, 16)] = (
                            rv[r, pl.ds(j * 16, 16)] * wk)
                return carry

            lax.fori_loop(0, CK // 2, _srow, 0)

        for k in range(4):
            _fill_slot(k, k)
        pltpu.sync_copy(wdata_hbm.at[wid], wdata_v)

        zero16 = jnp.zeros((16,), jnp.float32)

        def _zrow(r, carry):
            for j in range(H // 16):
                rows0[r, pl.ds(j * 16, 16)] = zero16
            return carry

        lax.fori_loop(0, CK, _zrow, 0)
        rows_per_sub = NP // NS
        for i in range(rows_per_sub // CK):
            pltpu.sync_copy(rows0,
                            agg_s.at[pl.ds(sid * rows_per_sub + i * CK, CK)])
        plsc.subcore_barrier()

        _wait_slot(0)
        _gather(0, 0)

        def _quad(i, carry):
            for p in range(4):
                c = 4 * i + p
                rp = p & 1

                if p == 0:
                    @pl.when(i > 0)
                    def _():
                        _wait_scatter(1 - rp)
                else:
                    _wait_scatter(1 - rp)
                _wait_slot((p + 1) % 4)
                _gather(1 - rp, (p + 1) % 4)

                _wait_gather(rp)
                for j in range(H // 16):
                    dstb[rp, pl.ds(j * 16, 16)] = ebuf[p, 1, pl.ds(j * 16, 16)]
                _fill_slot(p, c + 4)
                _scale(rp, c)
                pltpu.async_copy(rows[rp], agg_s.at[dstb.at[rp]], ssems[rp],
                                 add=True)
            return carry

        lax.fori_loop(0, NCHUNK // 4, _quad, 0)

        _wait_gather(0)
        _wait_scatter(1)
        for k in (1, 2, 3):
            _wait_slot(k)
        plsc.subcore_barrier()

        descs = []
        for i in range(rows_per_sub // CK):
            p = i % 2
            r0 = sid * rows_per_sub + i * CK
            if i >= 2:
                descs[i - 2].wait()
            pltpu.sync_copy(agg_s.at[pl.ds(r0, CK)], rows[p])
            descs.append(
                pltpu.async_copy(rows[p], out_hbm.at[pl.ds(cid * NP + r0, CK)],
                                 gsems[p]))
        for d in descs[-2:]:
            d.wait()

    return gather_k, edges_k


def _sc_gather(x3, embed):
    return _build_sc_kernels()[0](x3, embed)


def _sc_edges(edata, wdata, m):
    return _build_sc_kernels()[1](edata, wdata, m)


def _mm_body(h_ref, w_ref, o_ref):
    o_ref[...] = jnp.dot(h_ref[...], w_ref[...],
                         preferred_element_type=jnp.float32)


def _gru_body(h_ref, a0_ref, a1_ref, b3_ref, wih_ref, whh_ref, bih_ref,
              bhh_ref, v_ref, last_ref):
    i = pl.program_id(0)
    agg = a0_ref[...] + a1_ref[...]
    h = h_ref[...]
    gi = jax.lax.dot_general(agg, wih_ref[...], (((1,), (1,)), ((), ())),
                             preferred_element_type=jnp.float32) + bih_ref[...]
    gh = jax.lax.dot_general(h, whh_ref[...], (((1,), (1,)), ((), ())),
                             preferred_element_type=jnp.float32) + bhh_ref[...]
    r = jax.nn.sigmoid(gi[:, :H] + gh[:, :H])
    z = jax.nn.sigmoid(gi[:, H:2 * H] + gh[:, H:2 * H])
    n = jnp.tanh(gi[:, 2 * H:] + r * gh[:, 2 * H:])
    v_ref[...] = (1.0 - z) * n + z * h

    bb = b3_ref[0, 0, :]
    gid = i * RB + lax.broadcasted_iota(jnp.int32, (B, RB), 1)
    eq = bb[None, :] == lax.broadcasted_iota(jnp.int32, (B, RB), 0)
    cand = jnp.where(eq, gid, -1)
    bmax = jnp.max(cand, axis=1)[None, :]

    @pl.when(i == 0)
    def _():
        last_ref[...] = jnp.full((1, B), -1, jnp.int32)

    last_ref[...] = jnp.maximum(last_ref[...], bmax)


def _sl_body(v_ref, last_ref, o_ref):
    i = pl.program_id(0)
    lastv = jnp.maximum(last_ref[0, :], 0)
    gid = i * RB + lax.broadcasted_iota(jnp.int32, (B, RB), 1)
    oh = (lastv[:, None] == gid).astype(jnp.float32)
    part = jnp.dot(oh, v_ref[...], preferred_element_type=jnp.float32)

    @pl.when(i == 0)
    def _():
        o_ref[...] = jnp.zeros_like(o_ref)

    o_ref[...] += part


def _sg_body(v_ref, b3_ref, sl_ref, w1_ref, w2_ref, b12_ref, q_ref, qb_ref,
             o_ref):
    i = pl.program_id(0)
    bb = b3_ref[0, 0, :]
    v = v_ref[...]
    oh = (bb[:, None] == lax.broadcasted_iota(jnp.int32, (RB, B), 1)
          ).astype(jnp.float32)
    v_n = jnp.dot(oh, sl_ref[...], preferred_element_type=jnp.float32)
    pre = (jax.lax.dot_general(v_n, w1_ref[...], (((1,), (1,)), ((), ())),
                               preferred_element_type=jnp.float32)
           + jax.lax.dot_general(v, w2_ref[...], (((1,), (1,)), ((), ())),
                                 preferred_element_type=jnp.float32)
           + b12_ref[...])
    sig = jax.nn.sigmoid(pre)
    alpha = jnp.dot(sig, q_ref[...],
                    preferred_element_type=jnp.float32) + qb_ref[0, 0]
    contrib = alpha * v
    part = jax.lax.dot_general(oh, contrib, (((0,), (0,)), ((), ())),
                               preferred_element_type=jnp.float32)

    @pl.when(i == 0)
    def _():
        o_ref[...] = jnp.zeros_like(o_ref)

    o_ref[...] += part


def _sh_body(sl_ref, sg_ref, w3_ref, b3w_ref, o_ref):
    o_ref[...] = (
        jax.lax.dot_general(sl_ref[...], w3_ref[:, :H],
                            (((1,), (1,)), ((), ())),
                            preferred_element_type=jnp.float32)
        + jax.lax.dot_general(sg_ref[...], w3_ref[:, H:],
                              (((1,), (1,)), ((), ())),
                              preferred_element_type=jnp.float32)
        + b3w_ref[...])


def _z_body(sh_ref, emb_ref, o_ref):
    o_ref[...] = jax.lax.dot_general(emb_ref[...], sh_ref[...],
                                     (((1,), (1,)), ((), ())),
                                     preferred_element_type=jnp.float32)


def kernel(x, edge_index, edge_weight, batch, embed, W_ggc, W_ih, W_hh, b_ih,
           b_hh, W1_w, W1_b, W2_w, W2_b, q_w, q_b, W3_w, W3_b):
    xp = jnp.concatenate([x, jnp.zeros((NP - N,), x.dtype)])
    x3 = xp.astype(jnp.int32).reshape(NW, 4, 80)
    pe = NW * NCHUNK * CK
    pad_e = pe - E
    pad_idx = ((jnp.arange(pad_e, dtype=jnp.int32) * 97) % N)
    src_p = jnp.concatenate(
        [edge_index[0].astype(jnp.int32), pad_idx]).reshape(NW, NCHUNK, CK)
    dst_p = jnp.concatenate(
        [edge_index[1].astype(jnp.int32), pad_idx]).reshape(NW, NCHUNK, CK)
    w_p = jnp.concatenate([edge_weight, jnp.zeros((pad_e,), jnp.float32)])
    edata = jnp.stack([src_p, dst_p], axis=2)
    npad = CKP - NCHUNK
    pc_idx = ((jnp.arange(npad * CK, dtype=jnp.int32) * 131) % N
              ).reshape(npad, CK)
    pc = jnp.stack([pc_idx, pc_idx], axis=1)
    pc = jnp.broadcast_to(pc[None], (NW, npad, 2, CK))
    edata = jnp.concatenate([edata, pc], axis=1)
    wdata = jnp.concatenate(
        [w_p.reshape(NW, NCHUNK * CK),
         jnp.zeros((NW, npad * CK), jnp.float32)], axis=1)
    batch_p = jnp.concatenate(
        [batch.astype(jnp.int32), jnp.full((NP - N,), 2**30, jnp.int32)])
    batch3 = batch_p.reshape(NP // RB, 1, RB)
    bih2 = b_ih.reshape(1, 3 * H)
    bhh2 = b_hh.reshape(1, 3 * H)
    b12 = (W1_b + W2_b).reshape(1, H)
    qb2 = q_b.reshape(1, 1)
    qmat = jnp.broadcast_to(q_w.reshape(H, 1), (H, H))
    b3w = W3_b.reshape(1, H)

    h = _sc_gather(x3, embed)

    nb = NP // RB
    m = pl.pallas_call(
        _mm_body,
        grid=(nb,),
        in_specs=[pl.BlockSpec((RB, H), lambda i: (i, 0)),
                  pl.BlockSpec((H, H), lambda i: (0, 0))],
        out_specs=pl.BlockSpec((RB, H), lambda i: (i, 0)),
        out_shape=jax.ShapeDtypeStruct((NP, H), jnp.float32),
    )(h, W_ggc)

    agg2 = _sc_edges(edata, wdata, m)

    v, last = pl.pallas_call(
        _gru_body,
        grid=(nb,),
        in_specs=[
            pl.BlockSpec((RB, H), lambda i: (i, 0)),
            pl.BlockSpec((RB, H), lambda i: (i, 0)),
            pl.BlockSpec((RB, H), lambda i: (i + nb, 0)),
            pl.BlockSpec((1, 1, RB), lambda i: (i, 0, 0)),
            pl.BlockSpec((3 * H, H), lambda i: (0, 0)),
            pl.BlockSpec((3 * H, H), lambda i: (0, 0)),
            pl.BlockSpec((1, 3 * H), lambda i: (0, 0)),
            pl.BlockSpec((1, 3 * H), lambda i: (0, 0)),
        ],
        out_specs=[pl.BlockSpec((RB, H), lambda i: (i, 0)),
                   pl.BlockSpec((1, B), lambda i: (0, 0))],
        out_shape=[jax.ShapeDtypeStruct((NP, H), jnp.float32),
                   jax.ShapeDtypeStruct((1, B), jnp.int32)],
    )(h, agg2, agg2, batch3, W_ih, W_hh, bih2, bhh2)

    s_l = pl.pallas_call(
        _sl_body,
        grid=(nb,),
        in_specs=[pl.BlockSpec((RB, H), lambda i: (i, 0)),
                  pl.BlockSpec((1, B), lambda i: (0, 0))],
        out_specs=pl.BlockSpec((B, H), lambda i: (0, 0)),
        out_shape=jax.ShapeDtypeStruct((B, H), jnp.float32),
    )(v, last)

    s_g = pl.pallas_call(
        _sg_body,
        grid=(nb,),
        in_specs=[
            pl.BlockSpec((RB, H), lambda i: (i, 0)),
            pl.BlockSpec((1, 1, RB), lambda i: (i, 0, 0)),
            pl.BlockSpec((B, H), lambda i: (0, 0)),
            pl.BlockSpec((H, H), lambda i: (0, 0)),
            pl.BlockSpec((H, H), lambda i: (0, 0)),
            pl.BlockSpec((1, H), lambda i: (0, 0)),
            pl.BlockSpec((H, H), lambda i: (0, 0)),
            pl.BlockSpec(memory_space=pltpu.SMEM),
        ],
        out_specs=pl.BlockSpec((B, H), lambda i: (0, 0)),
        out_shape=jax.ShapeDtypeStruct((B, H), jnp.float32),
    )(v, batch3, s_l, W1_w, W2_w, b12, qmat, qb2)

    s_h = pl.pallas_call(
        _sh_body,
        in_specs=[pl.BlockSpec((B, H), lambda: (0, 0)),
                  pl.BlockSpec((B, H), lambda: (0, 0)),
                  pl.BlockSpec((H, 2 * H), lambda: (0, 0)),
                  pl.BlockSpec((1, H), lambda: (0, 0))],
        out_specs=pl.BlockSpec((B, H), lambda: (0, 0)),
        out_shape=jax.ShapeDtypeStruct((B, H), jnp.float32),
    )(s_l, s_g, W3_w, b3w)

    nvb = -(-NV // VB)
    zt = pl.pallas_call(
        _z_body,
        grid=(nvb,),
        in_specs=[pl.BlockSpec((B, H), lambda i: (0, 0)),
                  pl.BlockSpec((VB, H), lambda i: (i, 0))],
        out_specs=pl.BlockSpec((VB, B), lambda i: (i, 0)),
        out_shape=jax.ShapeDtypeStruct((NV, B), jnp.float32),
    )(s_h, embed)

    return zt.T


# speedup vs baseline: 8.2797x; 1.2598x over previous
import functools

import jax
import jax.numpy as jnp
from jax import lax
from jax.experimental import pallas as pl
from jax.experimental.pallas import tpu as pltpu
from jax.experimental.pallas import tpu_sc as plsc

N = 10000
E = 320000
NV = 100000
H = 128
B = 256

NC = 2
NS = 16
NW = NC * NS
NP = 10240
GPW = NP // NW
CK = 128
NCHUNK = 80
CKP = NCHUNK + 4
GCH = NW * NCHUNK + 4

RB = 1024
VB = 4096

@functools.cache
def _build_sc_kernels():
    mesh = plsc.VectorSubcoreMesh(core_axis_name="c", subcore_axis_name="s",
                                  num_cores=NC, num_subcores=NS)

    @functools.partial(
        pl.kernel, mesh=mesh,
        out_type=jax.ShapeDtypeStruct((NP, H), jnp.float32),
        scratch_types=[
            pltpu.VMEM((4, 80), jnp.int32),
            pltpu.VMEM((GPW, H), jnp.float32),
            pltpu.SemaphoreType.DMA,
        ],
    )
    def gather_k(x3_hbm, embed_hbm, out_hbm, idx_v, rows_v, sem):
        wid = lax.axis_index("s") * NC + lax.axis_index("c")
        pltpu.sync_copy(x3_hbm.at[wid], idx_v)
        descs = [
            pltpu.async_copy(embed_hbm.at[idx_v.at[i]],
                             rows_v.at[pl.ds(i * 80, 80)], sem)
            for i in range(4)
        ]
        for d in descs:
            d.wait()
        pltpu.sync_copy(rows_v, out_hbm.at[pl.ds(wid * GPW, GPW)])

    @functools.partial(
        pl.kernel, mesh=mesh,
        compiler_params=pltpu.CompilerParams(needs_layout_passes=False),
        out_type=jax.ShapeDtypeStruct((NC * NP, H), jnp.float32),
        scratch_types=(
            [pltpu.VMEM((4, 2, CK), jnp.int32),
             pltpu.VMEM((CKP * CK,), jnp.float32),
             pltpu.VMEM((2, CK), jnp.int32)]
            + [pltpu.VMEM((CK, H), jnp.float32)] * 2
            + [pltpu.VMEM_SHARED((NP, H), jnp.float32)]
            + [pltpu.SemaphoreType.DMA] * 8
        ),
    )
    def edges_k(sd_hbm, wdata_hbm, m_hbm, out_hbm,
                ebuf, wdata_v, dstb, rows0, rows1, agg_s,
                gs0, gs1, ss0, ss1, es0, es1, es2, es3):
        rows = (rows0, rows1)
        gsems = (gs0, gs1)
        ssems = (ss0, ss1)
        esems = (es0, es1, es2, es3)
        cid = lax.axis_index("c")
        sid = lax.axis_index("s")
        wid = sid * NC + cid

        def _fill_slot(slot, c):
            gc = wid * NCHUNK + c
            pltpu.async_copy(sd_hbm.at[0, gc], ebuf.at[slot, 0], esems[slot])
            pltpu.async_copy(sd_hbm.at[1, gc], ebuf.at[slot, 1], esems[slot])

        def _wait_slot(slot):
            for k in range(2):
                pltpu.make_async_copy(sd_hbm.at[0, 0], ebuf.at[slot, k],
                                      esems[slot]).wait()

        def _gather(rp, slot):
            pltpu.async_copy(m_hbm.at[ebuf.at[slot, 0]], rows[rp], gsems[rp])

        def _wait_gather(rp):
            pltpu.make_async_copy(m_hbm.at[pl.ds(0, CK)], rows[rp],
                                  gsems[rp]).wait()

        def _wait_scatter(rp):
            pltpu.make_async_copy(rows[rp], agg_s.at[pl.ds(0, CK)],
                                  ssems[rp]).wait()

        def _scale(rp, c):
            rv = rows[rp]
            cbase = c * CK

            @plsc.parallel_loop(0, CK, 1, unroll=4)
            def _srow(r):
                wk = plsc.load_gather(
                    wdata_v, [jnp.full((16,), cbase + r, jnp.int32)])
                for j in range(H // 16):
                    rv[r, pl.ds(j * 16, 16)] = rv[r, pl.ds(j * 16, 16)] * wk

        for k in range(4):
            _fill_slot(k, k)
        pltpu.sync_copy(wdata_hbm.at[pl.ds(wid * NCHUNK * CK, CKP * CK)],
                        wdata_v)

        zero16 = jnp.zeros((16,), jnp.float32)

        def _zrow(r, carry):
            for j in range(H // 16):
                rows0[r, pl.ds(j * 16, 16)] = zero16
            return carry

        lax.fori_loop(0, CK, _zrow, 0)
        rows_per_sub = NP // NS
        for i in range(rows_per_sub // CK):
            pltpu.sync_copy(rows0,
                            agg_s.at[pl.ds(sid * rows_per_sub + i * CK, CK)])
        plsc.subcore_barrier()

        _wait_slot(0)
        _gather(0, 0)

        def _quad(i, carry):
            for p in range(4):
                c = 4 * i + p
                rp = p & 1

                if p == 0:
                    @pl.when(i > 0)
                    def _():
                        _wait_scatter(1 - rp)
                else:
                    _wait_scatter(1 - rp)
                _wait_slot((p + 1) % 4)
                _gather(1 - rp, (p + 1) % 4)

                _wait_gather(rp)
                for j in range(H // 16):
                    dstb[rp, pl.ds(j * 16, 16)] = ebuf[p, 1, pl.ds(j * 16, 16)]
                _fill_slot(p, c + 4)
                _scale(rp, c)
                pltpu.async_copy(rows[rp], agg_s.at[dstb.at[rp]], ssems[rp],
                                 add=True)
            return carry

        lax.fori_loop(0, NCHUNK // 4, _quad, 0)

        _wait_gather(0)
        _wait_scatter(1)
        for k in (1, 2, 3):
            _wait_slot(k)
        plsc.subcore_barrier()

        descs = []
        for i in range(rows_per_sub // CK):
            p = i % 2
            r0 = sid * rows_per_sub + i * CK
            if i >= 2:
                descs[i - 2].wait()
            pltpu.sync_copy(agg_s.at[pl.ds(r0, CK)], rows[p])
            descs.append(
                pltpu.async_copy(rows[p], out_hbm.at[pl.ds(cid * NP + r0, CK)],
                                 gsems[p]))
        for d in descs[-2:]:
            d.wait()

    return gather_k, edges_k


def _sc_gather(x3, embed):
    return _build_sc_kernels()[0](x3, embed)


def _sc_edges(edata, wdata, m):
    return _build_sc_kernels()[1](edata, wdata, m)


def _mm_body(h_ref, w_ref, o_ref):
    o_ref[...] = jnp.dot(h_ref[...], w_ref[...],
                         preferred_element_type=jnp.float32)


def _last_body(b3_ref, last_ref):
    i = pl.program_id(0)
    bb = b3_ref[0, 0, :]
    gid = i * RB + lax.broadcasted_iota(jnp.int32, (B, RB), 1)
    eq = bb[None, :] == lax.broadcasted_iota(jnp.int32, (B, RB), 0)
    cand = jnp.where(eq, gid, -1)
    bmax = jnp.max(cand, axis=1)[None, :]

    @pl.when(i == 0)
    def _():
        last_ref[...] = jnp.full((1, B), -1, jnp.int32)

    last_ref[...] = jnp.maximum(last_ref[...], bmax)


def _gru_body(h_ref, a0_ref, a1_ref, last_ref, wih_ref, whh_ref, bih_ref,
              bhh_ref, v_ref, sl_ref):
    i = pl.program_id(0)
    agg = a0_ref[...] + a1_ref[...]
    h = h_ref[...]
    gi = jax.lax.dot_general(agg, wih_ref[...], (((1,), (1,)), ((), ())),
                             preferred_element_type=jnp.float32) + bih_ref[...]
    gh = jax.lax.dot_general(h, whh_ref[...], (((1,), (1,)), ((), ())),
                             preferred_element_type=jnp.float32) + bhh_ref[...]
    r = jax.nn.sigmoid(gi[:, :H] + gh[:, :H])
    z = jax.nn.sigmoid(gi[:, H:2 * H] + gh[:, H:2 * H])
    n = jnp.tanh(gi[:, 2 * H:] + r * gh[:, 2 * H:])
    v = (1.0 - z) * n + z * h
    v_ref[...] = v

    lastv = jnp.maximum(last_ref[0, :], 0)
    gid = i * RB + lax.broadcasted_iota(jnp.int32, (B, RB), 1)
    oh = (lastv[:, None] == gid).astype(jnp.float32)
    part = jnp.dot(oh, v, preferred_element_type=jnp.float32)

    @pl.when(i == 0)
    def _():
        sl_ref[...] = jnp.zeros_like(sl_ref)

    sl_ref[...] += part


def _sg_body(v_ref, b3_ref, sl_ref, w1_ref, w2_ref, b12_ref, q_ref, qb_ref,
             w3_ref, b3w_ref, o_ref, sh_ref):
    i = pl.program_id(0)
    bb = b3_ref[0, 0, :]
    v = v_ref[...]
    oh = (bb[:, None] == lax.broadcasted_iota(jnp.int32, (RB, B), 1)
          ).astype(jnp.float32)
    v_n = jnp.dot(oh, sl_ref[...], preferred_element_type=jnp.float32)
    pre = (jax.lax.dot_general(v_n, w1_ref[...], (((1,), (1,)), ((), ())),
                               preferred_element_type=jnp.float32)
           + jax.lax.dot_general(v, w2_ref[...], (((1,), (1,)), ((), ())),
                                 preferred_element_type=jnp.float32)
           + b12_ref[...])
    sig = jax.nn.sigmoid(pre)
    alpha = jnp.dot(sig, q_ref[...],
                    preferred_element_type=jnp.float32) + qb_ref[0, 0]
    contrib = alpha * v
    part = jax.lax.dot_general(oh, contrib, (((0,), (0,)), ((), ())),
                               preferred_element_type=jnp.float32)

    @pl.when(i == 0)
    def _():
        o_ref[...] = jnp.zeros_like(o_ref)

    o_ref[...] += part

    @pl.when(i == pl.num_programs(0) - 1)
    def _():
        sh_ref[...] = (
            jax.lax.dot_general(sl_ref[...], w3_ref[:, :H],
                                (((1,), (1,)), ((), ())),
                                preferred_element_type=jnp.float32)
            + jax.lax.dot_general(o_ref[...], w3_ref[:, H:],
                                  (((1,), (1,)), ((), ())),
                                  preferred_element_type=jnp.float32)
            + b3w_ref[...])


def _z_body(sh_ref, emb_ref, o_ref):
    o_ref[...] = jax.lax.dot_general(emb_ref[...], sh_ref[...],
                                     (((1,), (1,)), ((), ())),
                                     preferred_element_type=jnp.float32)


def kernel(x, edge_index, edge_weight, batch, embed, W_ggc, W_ih, W_hh, b_ih,
           b_hh, W1_w, W1_b, W2_w, W2_b, q_w, q_b, W3_w, W3_b):
    xp = jnp.concatenate([x, jnp.zeros((NP - N,), x.dtype)])
    x3 = xp.astype(jnp.int32).reshape(NW, 4, 80)
    pad_e = GCH * CK - E
    pad_idx = ((jnp.arange(pad_e, dtype=jnp.int32) * 97) % N)
    sd = jnp.concatenate(
        [edge_index.astype(jnp.int32), jnp.stack([pad_idx, pad_idx])],
        axis=1).reshape(2, GCH, CK)
    wdata = jnp.concatenate(
        [edge_weight, jnp.zeros((pad_e,), jnp.float32)])
    batch_p = jnp.concatenate(
        [batch.astype(jnp.int32), jnp.full((NP - N,), 2**30, jnp.int32)])
    batch3 = batch_p.reshape(NP // RB, 1, RB)
    bih2 = b_ih.reshape(1, 3 * H)
    bhh2 = b_hh.reshape(1, 3 * H)
    b12 = (W1_b + W2_b).reshape(1, H)
    qb2 = q_b.reshape(1, 1)
    qmat = jnp.broadcast_to(q_w.reshape(H, 1), (H, H))
    b3w = W3_b.reshape(1, H)

    h = _sc_gather(x3, embed)

    nb = NP // RB
    m = pl.pallas_call(
        _mm_body,
        grid=(nb,),
        in_specs=[pl.BlockSpec((RB, H), lambda i: (i, 0)),
                  pl.BlockSpec((H, H), lambda i: (0, 0))],
        out_specs=pl.BlockSpec((RB, H), lambda i: (i, 0)),
        out_shape=jax.ShapeDtypeStruct((NP, H), jnp.float32),
    )(h, W_ggc)

    last = pl.pallas_call(
        _last_body,
        grid=(nb,),
        in_specs=[pl.BlockSpec((1, 1, RB), lambda i: (i, 0, 0))],
        out_specs=pl.BlockSpec((1, B), lambda i: (0, 0)),
        out_shape=jax.ShapeDtypeStruct((1, B), jnp.int32),
    )(batch3)

    agg2 = _sc_edges(sd, wdata, m)

    v, s_l = pl.pallas_call(
        _gru_body,
        grid=(nb,),
        in_specs=[
            pl.BlockSpec((RB, H), lambda i: (i, 0)),
            pl.BlockSpec((RB, H), lambda i: (i, 0)),
            pl.BlockSpec((RB, H), lambda i: (i + nb, 0)),
            pl.BlockSpec((1, B), lambda i: (0, 0)),
            pl.BlockSpec((3 * H, H), lambda i: (0, 0)),
            pl.BlockSpec((3 * H, H), lambda i: (0, 0)),
            pl.BlockSpec((1, 3 * H), lambda i: (0, 0)),
            pl.BlockSpec((1, 3 * H), lambda i: (0, 0)),
        ],
        out_specs=[pl.BlockSpec((RB, H), lambda i: (i, 0)),
                   pl.BlockSpec((B, H), lambda i: (0, 0))],
        out_shape=[jax.ShapeDtypeStruct((NP, H), jnp.float32),
                   jax.ShapeDtypeStruct((B, H), jnp.float32)],
    )(h, agg2, agg2, last, W_ih, W_hh, bih2, bhh2)

    s_g, s_h = pl.pallas_call(
        _sg_body,
        grid=(nb,),
        in_specs=[
            pl.BlockSpec((RB, H), lambda i: (i, 0)),
            pl.BlockSpec((1, 1, RB), lambda i: (i, 0, 0)),
            pl.BlockSpec((B, H), lambda i: (0, 0)),
            pl.BlockSpec((H, H), lambda i: (0, 0)),
            pl.BlockSpec((H, H), lambda i: (0, 0)),
            pl.BlockSpec((1, H), lambda i: (0, 0)),
            pl.BlockSpec((H, H), lambda i: (0, 0)),
            pl.BlockSpec(memory_space=pltpu.SMEM),
            pl.BlockSpec((H, 2 * H), lambda i: (0, 0)),
            pl.BlockSpec((1, H), lambda i: (0, 0)),
        ],
        out_specs=[pl.BlockSpec((B, H), lambda i: (0, 0)),
                   pl.BlockSpec((B, H), lambda i: (0, 0))],
        out_shape=[jax.ShapeDtypeStruct((B, H), jnp.float32),
                   jax.ShapeDtypeStruct((B, H), jnp.float32)],
    )(v, batch3, s_l, W1_w, W2_w, b12, qmat, qb2, W3_w, b3w)
    del s_g

    nvb = -(-NV // VB)
    zt = pl.pallas_call(
        _z_body,
        grid=(nvb,),
        in_specs=[pl.BlockSpec((B, H), lambda i: (0, 0)),
                  pl.BlockSpec((VB, H), lambda i: (i, 0))],
        out_specs=pl.BlockSpec((VB, B), lambda i: (i, 0)),
        out_shape=jax.ShapeDtypeStruct((NV, B), jnp.float32),
    )(s_h, embed)

    return zt.T


# speedup vs baseline: 8.6441x; 1.0440x over previous
import functools

import jax
import jax.numpy as jnp
from jax import lax
from jax.experimental import pallas as pl
from jax.experimental.pallas import tpu as pltpu
from jax.experimental.pallas import tpu_sc as plsc

N = 10000
E = 320000
NV = 100000
H = 128
B = 256

NC = 2
NS = 16
NW = NC * NS
NP = 10240
GPW = NP // NW
CK = 128
NCHUNK = 80
CKP = NCHUNK + 4
GCH = NW * NCHUNK + 4

RB = 2048
VB = 4096

@functools.cache
def _build_sc_kernels():
    mesh = plsc.VectorSubcoreMesh(core_axis_name="c", subcore_axis_name="s",
                                  num_cores=NC, num_subcores=NS)

    @functools.partial(
        pl.kernel, mesh=mesh,
        out_type=jax.ShapeDtypeStruct((NP, H), jnp.float32),
        scratch_types=[
            pltpu.VMEM((4, 80), jnp.int32),
            pltpu.VMEM((GPW, H), jnp.float32),
            pltpu.SemaphoreType.DMA,
        ],
    )
    def gather_k(x3_hbm, embed_hbm, out_hbm, idx_v, rows_v, sem):
        wid = lax.axis_index("s") * NC + lax.axis_index("c")
        pltpu.sync_copy(x3_hbm.at[wid], idx_v)
        descs = [
            pltpu.async_copy(embed_hbm.at[idx_v.at[i]],
                             rows_v.at[pl.ds(i * 80, 80)], sem)
            for i in range(4)
        ]
        for d in descs:
            d.wait()
        pltpu.sync_copy(rows_v, out_hbm.at[pl.ds(wid * GPW, GPW)])

    @functools.partial(
        pl.kernel, mesh=mesh,
        compiler_params=pltpu.CompilerParams(needs_layout_passes=False),
        out_type=jax.ShapeDtypeStruct((NC * NP, H), jnp.float32),
        scratch_types=(
            [pltpu.VMEM((4, 2, CK), jnp.int32),
             pltpu.VMEM((CKP * CK,), jnp.float32),
             pltpu.VMEM((2, CK), jnp.int32)]
            + [pltpu.VMEM((CK, H), jnp.float32)] * 2
            + [pltpu.VMEM_SHARED((NP, H), jnp.float32)]
            + [pltpu.SemaphoreType.DMA] * 8
        ),
    )
    def edges_k(sd_hbm, wdata_hbm, m_hbm, out_hbm,
                ebuf, wdata_v, dstb, rows0, rows1, agg_s,
                gs0, gs1, ss0, ss1, es0, es1, es2, es3):
        rows = (rows0, rows1)
        gsems = (gs0, gs1)
        ssems = (ss0, ss1)
        esems = (es0, es1, es2, es3)
        cid = lax.axis_index("c")
        sid = lax.axis_index("s")
        wid = sid * NC + cid

        def _fill_slot(slot, c):
            gc = wid * NCHUNK + c
            pltpu.async_copy(sd_hbm.at[0, gc], ebuf.at[slot, 0], esems[slot])
            pltpu.async_copy(sd_hbm.at[1, gc], ebuf.at[slot, 1], esems[slot])

        def _wait_slot(slot):
            for k in range(2):
                pltpu.make_async_copy(sd_hbm.at[0, 0], ebuf.at[slot, k],
                                      esems[slot]).wait()

        def _gather(rp, slot):
            pltpu.async_copy(m_hbm.at[ebuf.at[slot, 0]], rows[rp], gsems[rp])

        def _wait_gather(rp):
            pltpu.make_async_copy(m_hbm.at[pl.ds(0, CK)], rows[rp],
                                  gsems[rp]).wait()

        def _wait_scatter(rp):
            pltpu.make_async_copy(rows[rp], agg_s.at[pl.ds(0, CK)],
                                  ssems[rp]).wait()

        def _scale(rp, c):
            rv = rows[rp]
            cbase = c * CK

            @plsc.parallel_loop(0, CK, 1, unroll=4)
            def _srow(r):
                wk = plsc.load_gather(
                    wdata_v, [jnp.full((16,), cbase + r, jnp.int32)])
                for j in range(H // 16):
                    rv[r, pl.ds(j * 16, 16)] = rv[r, pl.ds(j * 16, 16)] * wk

        for k in range(4):
            _fill_slot(k, k)
        pltpu.sync_copy(wdata_hbm.at[pl.ds(wid * NCHUNK * CK, CKP * CK)],
                        wdata_v)

        zero16 = jnp.zeros((16,), jnp.float32)

        def _zrow(r, carry):
            for j in range(H // 16):
                rows0[r, pl.ds(j * 16, 16)] = zero16
            return carry

        lax.fori_loop(0, CK, _zrow, 0)
        rows_per_sub = NP // NS
        for i in range(rows_per_sub // CK):
            pltpu.sync_copy(rows0,
                            agg_s.at[pl.ds(sid * rows_per_sub + i * CK, CK)])
        plsc.subcore_barrier()

        _wait_slot(0)
        _gather(0, 0)

        def _quad(i, carry):
            for p in range(4):
                c = 4 * i + p
                rp = p & 1

                if p == 0:
                    @pl.when(i > 0)
                    def _():
                        _wait_scatter(1 - rp)
                else:
                    _wait_scatter(1 - rp)
                _wait_slot((p + 1) % 4)
                _gather(1 - rp, (p + 1) % 4)

                _wait_gather(rp)
                for j in range(H // 16):
                    dstb[rp, pl.ds(j * 16, 16)] = ebuf[p, 1, pl.ds(j * 16, 16)]
                _fill_slot(p, c + 4)
                _scale(rp, c)
                pltpu.async_copy(rows[rp], agg_s.at[dstb.at[rp]], ssems[rp],
                                 add=True)
            return carry

        lax.fori_loop(0, NCHUNK // 4, _quad, 0)

        _wait_gather(0)
        _wait_scatter(1)
        for k in (1, 2, 3):
            _wait_slot(k)
        plsc.subcore_barrier()

        descs = []
        for i in range(rows_per_sub // CK):
            p = i % 2
            r0 = sid * rows_per_sub + i * CK
            if i >= 2:
                descs[i - 2].wait()
            pltpu.sync_copy(agg_s.at[pl.ds(r0, CK)], rows[p])
            descs.append(
                pltpu.async_copy(rows[p], out_hbm.at[pl.ds(cid * NP + r0, CK)],
                                 gsems[p]))
        for d in descs[-2:]:
            d.wait()

    return gather_k, edges_k


def _sc_gather(x3, embed):
    return _build_sc_kernels()[0](x3, embed)


def _sc_edges(edata, wdata, m):
    return _build_sc_kernels()[1](edata, wdata, m)


def _mm_body(h_ref, w_ref, o_ref):
    o_ref[...] = jnp.dot(h_ref[...], w_ref[...],
                         preferred_element_type=jnp.float32)


def _last_body(b3_ref, last_ref):
    i = pl.program_id(0)
    bb = b3_ref[0, 0, :]
    gid = i * RB + lax.broadcasted_iota(jnp.int32, (B, RB), 1)
    eq = bb[None, :] == lax.broadcasted_iota(jnp.int32, (B, RB), 0)
    cand = jnp.where(eq, gid, -1)
    bmax = jnp.max(cand, axis=1)[None, :]

    @pl.when(i == 0)
    def _():
        last_ref[...] = jnp.full((1, B), -1, jnp.int32)

    last_ref[...] = jnp.maximum(last_ref[...], bmax)


def _gru_body(h_ref, a0_ref, a1_ref, last_ref, wih_ref, whh_ref, bih_ref,
              bhh_ref, v_ref, sl_ref):
    i = pl.program_id(0)
    agg = a0_ref[...] + a1_ref[...]
    h = h_ref[...]
    gi = jax.lax.dot_general(agg, wih_ref[...], (((1,), (1,)), ((), ())),
                             preferred_element_type=jnp.float32) + bih_ref[...]
    gh = jax.lax.dot_general(h, whh_ref[...], (((1,), (1,)), ((), ())),
                             preferred_element_type=jnp.float32) + bhh_ref[...]
    r = jax.nn.sigmoid(gi[:, :H] + gh[:, :H])
    z = jax.nn.sigmoid(gi[:, H:2 * H] + gh[:, H:2 * H])
    n = jnp.tanh(gi[:, 2 * H:] + r * gh[:, 2 * H:])
    v = (1.0 - z) * n + z * h
    v_ref[...] = v

    lastv = jnp.maximum(last_ref[0, :], 0)
    gid = i * RB + lax.broadcasted_iota(jnp.int32, (B, RB), 1)
    oh = (lastv[:, None] == gid).astype(jnp.float32)
    part = jnp.dot(oh, v, preferred_element_type=jnp.float32)

    @pl.when(i == 0)
    def _():
        sl_ref[...] = jnp.zeros_like(sl_ref)

    sl_ref[...] += part


def _sg_body(v_ref, b3_ref, sl_ref, w1_ref, w2_ref, b12_ref, q_ref, qb_ref,
             w3_ref, b3w_ref, o_ref, sh_ref):
    i = pl.program_id(0)
    bb = b3_ref[0, 0, :]
    v = v_ref[...]
    oh = (bb[:, None] == lax.broadcasted_iota(jnp.int32, (RB, B), 1)
          ).astype(jnp.float32)
    v_n = jnp.dot(oh, sl_ref[...], preferred_element_type=jnp.float32)
    pre = (jax.lax.dot_general(v_n, w1_ref[...], (((1,), (1,)), ((), ())),
                               preferred_element_type=jnp.float32)
           + jax.lax.dot_general(v, w2_ref[...], (((1,), (1,)), ((), ())),
                                 preferred_element_type=jnp.float32)
           + b12_ref[...])
    sig = jax.nn.sigmoid(pre)
    alpha = jnp.dot(sig, q_ref[...],
                    preferred_element_type=jnp.float32) + qb_ref[0, 0]
    contrib = alpha * v
    part = jax.lax.dot_general(oh, contrib, (((0,), (0,)), ((), ())),
                               preferred_element_type=jnp.float32)

    @pl.when(i == 0)
    def _():
        o_ref[...] = jnp.zeros_like(o_ref)

    o_ref[...] += part

    @pl.when(i == pl.num_programs(0) - 1)
    def _():
        sh_ref[...] = (
            jax.lax.dot_general(sl_ref[...], w3_ref[:, :H],
                                (((1,), (1,)), ((), ())),
                                preferred_element_type=jnp.float32)
            + jax.lax.dot_general(o_ref[...], w3_ref[:, H:],
                                  (((1,), (1,)), ((), ())),
                                  preferred_element_type=jnp.float32)
            + b3w_ref[...])


def _z_body(sh_ref, emb_ref, o_ref):
    o_ref[...] = jax.lax.dot_general(emb_ref[...], sh_ref[...],
                                     (((1,), (1,)), ((), ())),
                                     preferred_element_type=jnp.float32)


def kernel(x, edge_index, edge_weight, batch, embed, W_ggc, W_ih, W_hh, b_ih,
           b_hh, W1_w, W1_b, W2_w, W2_b, q_w, q_b, W3_w, W3_b):
    xp = jnp.concatenate([x, jnp.zeros((NP - N,), x.dtype)])
    x3 = xp.astype(jnp.int32).reshape(NW, 4, 80)
    pad_e = GCH * CK - E
    pad_idx = ((jnp.arange(pad_e, dtype=jnp.int32) * 97) % N)
    sd = jnp.concatenate(
        [edge_index.astype(jnp.int32), jnp.stack([pad_idx, pad_idx])],
        axis=1).reshape(2, GCH, CK)
    wdata = jnp.concatenate(
        [edge_weight, jnp.zeros((pad_e,), jnp.float32)])
    batch_p = jnp.concatenate(
        [batch.astype(jnp.int32), jnp.full((NP - N,), 2**30, jnp.int32)])
    batch3 = batch_p.reshape(NP // RB, 1, RB)
    bih2 = b_ih.reshape(1, 3 * H)
    bhh2 = b_hh.reshape(1, 3 * H)
    b12 = (W1_b + W2_b).reshape(1, H)
    qb2 = q_b.reshape(1, 1)
    qmat = jnp.broadcast_to(q_w.reshape(H, 1), (H, H))
    b3w = W3_b.reshape(1, H)

    h = _sc_gather(x3, embed)

    nb = NP // RB
    m = pl.pallas_call(
        _mm_body,
        grid=(nb,),
        in_specs=[pl.BlockSpec((RB, H), lambda i: (i, 0)),
                  pl.BlockSpec((H, H), lambda i: (0, 0))],
        out_specs=pl.BlockSpec((RB, H), lambda i: (i, 0)),
        out_shape=jax.ShapeDtypeStruct((NP, H), jnp.float32),
    )(h, W_ggc)

    last = pl.pallas_call(
        _last_body,
        grid=(nb,),
        in_specs=[pl.BlockSpec((1, 1, RB), lambda i: (i, 0, 0))],
        out_specs=pl.BlockSpec((1, B), lambda i: (0, 0)),
        out_shape=jax.ShapeDtypeStruct((1, B), jnp.int32),
    )(batch3)

    agg2 = _sc_edges(sd, wdata, m)

    v, s_l = pl.pallas_call(
        _gru_body,
        grid=(nb,),
        in_specs=[
            pl.BlockSpec((RB, H), lambda i: (i, 0)),
            pl.BlockSpec((RB, H), lambda i: (i, 0)),
            pl.BlockSpec((RB, H), lambda i: (i + nb, 0)),
            pl.BlockSpec((1, B), lambda i: (0, 0)),
            pl.BlockSpec((3 * H, H), lambda i: (0, 0)),
            pl.BlockSpec((3 * H, H), lambda i: (0, 0)),
            pl.BlockSpec((1, 3 * H), lambda i: (0, 0)),
            pl.BlockSpec((1, 3 * H), lambda i: (0, 0)),
        ],
        out_specs=[pl.BlockSpec((RB, H), lambda i: (i, 0)),
                   pl.BlockSpec((B, H), lambda i: (0, 0))],
        out_shape=[jax.ShapeDtypeStruct((NP, H), jnp.float32),
                   jax.ShapeDtypeStruct((B, H), jnp.float32)],
    )(h, agg2, agg2, last, W_ih, W_hh, bih2, bhh2)

    s_g, s_h = pl.pallas_call(
        _sg_body,
        grid=(nb,),
        in_specs=[
            pl.BlockSpec((RB, H), lambda i: (i, 0)),
            pl.BlockSpec((1, 1, RB), lambda i: (i, 0, 0)),
            pl.BlockSpec((B, H), lambda i: (0, 0)),
            pl.BlockSpec((H, H), lambda i: (0, 0)),
            pl.BlockSpec((H, H), lambda i: (0, 0)),
            pl.BlockSpec((1, H), lambda i: (0, 0)),
            pl.BlockSpec((H, H), lambda i: (0, 0)),
            pl.BlockSpec(memory_space=pltpu.SMEM),
            pl.BlockSpec((H, 2 * H), lambda i: (0, 0)),
            pl.BlockSpec((1, H), lambda i: (0, 0)),
        ],
        out_specs=[pl.BlockSpec((B, H), lambda i: (0, 0)),
                   pl.BlockSpec((B, H), lambda i: (0, 0))],
        out_shape=[jax.ShapeDtypeStruct((B, H), jnp.float32),
                   jax.ShapeDtypeStruct((B, H), jnp.float32)],
    )(v, batch3, s_l, W1_w, W2_w, b12, qmat, qb2, W3_w, b3w)
    del s_g

    nvb = -(-NV // VB)
    zt = pl.pallas_call(
        _z_body,
        grid=(nvb,),
        in_specs=[pl.BlockSpec((B, H), lambda i: (0, 0)),
                  pl.BlockSpec((VB, H), lambda i: (i, 0))],
        out_specs=pl.BlockSpec((VB, B), lambda i: (i, 0)),
        out_shape=jax.ShapeDtypeStruct((NV, B), jnp.float32),
    )(s_h, embed)

    return zt.T


# speedup vs baseline: 8.9558x; 1.0361x over previous
import functools

import jax
import jax.numpy as jnp
from jax import lax
from jax.experimental import pallas as pl
from jax.experimental.pallas import tpu as pltpu
from jax.experimental.pallas import tpu_sc as plsc

N = 10000
E = 320000
NV = 100000
H = 128
B = 256

NC = 2
NS = 16
NW = NC * NS
NP = 10240
GPW = NP // NW
CK = 128
NCHUNK = 80
CKP = NCHUNK + 4
GCH = NW * NCHUNK + 4

RB = 2048
VB = 8192

@functools.cache
def _build_sc_kernels():
    mesh = plsc.VectorSubcoreMesh(core_axis_name="c", subcore_axis_name="s",
                                  num_cores=NC, num_subcores=NS)

    @functools.partial(
        pl.kernel, mesh=mesh,
        out_type=jax.ShapeDtypeStruct((NP, H), jnp.float32),
        scratch_types=[
            pltpu.VMEM((4, 80), jnp.int32),
            pltpu.VMEM((GPW, H), jnp.float32),
            pltpu.SemaphoreType.DMA,
        ],
    )
    def gather_k(x3_hbm, embed_hbm, out_hbm, idx_v, rows_v, sem):
        wid = lax.axis_index("s") * NC + lax.axis_index("c")
        pltpu.sync_copy(x3_hbm.at[wid], idx_v)
        descs = [
            pltpu.async_copy(embed_hbm.at[idx_v.at[i]],
                             rows_v.at[pl.ds(i * 80, 80)], sem)
            for i in range(4)
        ]
        for d in descs:
            d.wait()
        pltpu.sync_copy(rows_v, out_hbm.at[pl.ds(wid * GPW, GPW)])

    @functools.partial(
        pl.kernel, mesh=mesh,
        compiler_params=pltpu.CompilerParams(needs_layout_passes=False),
        out_type=jax.ShapeDtypeStruct((NC * NP, H), jnp.float32),
        scratch_types=(
            [pltpu.VMEM((4, 2, CK), jnp.int32),
             pltpu.VMEM((CKP * CK,), jnp.float32),
             pltpu.VMEM((2, CK), jnp.int32)]
            + [pltpu.VMEM((CK, H), jnp.float32)] * 2
            + [pltpu.VMEM_SHARED((NP, H), jnp.float32)]
            + [pltpu.SemaphoreType.DMA] * 8
        ),
    )
    def edges_k(sd_hbm, wdata_hbm, m_hbm, out_hbm,
                ebuf, wdata_v, dstb, rows0, rows1, agg_s,
                gs0, gs1, ss0, ss1, es0, es1, es2, es3):
        rows = (rows0, rows1)
        gsems = (gs0, gs1)
        ssems = (ss0, ss1)
        esems = (es0, es1, es2, es3)
        cid = lax.axis_index("c")
        sid = lax.axis_index("s")
        wid = sid * NC + cid

        def _fill_slot(slot, c):
            gc = wid * NCHUNK + c
            pltpu.async_copy(sd_hbm.at[0, gc], ebuf.at[slot, 0], esems[slot])
            pltpu.async_copy(sd_hbm.at[1, gc], ebuf.at[slot, 1], esems[slot])

        def _wait_slot(slot):
            for k in range(2):
                pltpu.make_async_copy(sd_hbm.at[0, 0], ebuf.at[slot, k],
                                      esems[slot]).wait()

        def _gather(rp, slot):
            pltpu.async_copy(m_hbm.at[ebuf.at[slot, 0]], rows[rp], gsems[rp])

        def _wait_gather(rp):
            pltpu.make_async_copy(m_hbm.at[pl.ds(0, CK)], rows[rp],
                                  gsems[rp]).wait()

        def _wait_scatter(rp):
            pltpu.make_async_copy(rows[rp], agg_s.at[pl.ds(0, CK)],
                                  ssems[rp]).wait()

        def _scale(rp, c):
            rv = rows[rp]
            cbase = c * CK

            @plsc.parallel_loop(0, CK, 1, unroll=4)
            def _srow(r):
                wk = plsc.load_gather(
                    wdata_v, [jnp.full((16,), cbase + r, jnp.int32)])
                for j in range(H // 16):
                    rv[r, pl.ds(j * 16, 16)] = rv[r, pl.ds(j * 16, 16)] * wk

        for k in range(4):
            _fill_slot(k, k)
        pltpu.sync_copy(wdata_hbm.at[pl.ds(wid * NCHUNK * CK, CKP * CK)],
                        wdata_v)

        zero16 = jnp.zeros((16,), jnp.float32)

        def _zrow(r, carry):
            for j in range(H // 16):
                rows0[r, pl.ds(j * 16, 16)] = zero16
            return carry

        lax.fori_loop(0, CK, _zrow, 0)
        rows_per_sub = NP // NS
        for i in range(rows_per_sub // CK):
            pltpu.sync_copy(rows0,
                            agg_s.at[pl.ds(sid * rows_per_sub + i * CK, CK)])
        plsc.subcore_barrier()

        _wait_slot(0)
        _gather(0, 0)

        def _quad(i, carry):
            for p in range(4):
                c = 4 * i + p
                rp = p & 1

                if p == 0:
                    @pl.when(i > 0)
                    def _():
                        _wait_scatter(1 - rp)
                else:
                    _wait_scatter(1 - rp)
                _wait_slot((p + 1) % 4)
                _gather(1 - rp, (p + 1) % 4)

                _wait_gather(rp)
                for j in range(H // 16):
                    dstb[rp, pl.ds(j * 16, 16)] = ebuf[p, 1, pl.ds(j * 16, 16)]
                _fill_slot(p, c + 4)
                _scale(rp, c)
                pltpu.async_copy(rows[rp], agg_s.at[dstb.at[rp]], ssems[rp],
                                 add=True)
            return carry

        lax.fori_loop(0, NCHUNK // 4, _quad, 0)

        _wait_gather(0)
        _wait_scatter(1)
        for k in (1, 2, 3):
            _wait_slot(k)
        plsc.subcore_barrier()

        descs = []
        for i in range(rows_per_sub // CK):
            p = i % 2
            r0 = sid * rows_per_sub + i * CK
            if i >= 2:
                descs[i - 2].wait()
            pltpu.sync_copy(agg_s.at[pl.ds(r0, CK)], rows[p])
            descs.append(
                pltpu.async_copy(rows[p], out_hbm.at[pl.ds(cid * NP + r0, CK)],
                                 gsems[p]))
        for d in descs[-2:]:
            d.wait()

    return gather_k, edges_k


def _sc_gather(x3, embed):
    return _build_sc_kernels()[0](x3, embed)


def _sc_edges(edata, wdata, m):
    return _build_sc_kernels()[1](edata, wdata, m)


def _last_body(b3_ref, last_ref):
    i = pl.program_id(0)
    bb = b3_ref[0, 0, :]
    gid = i * RB + lax.broadcasted_iota(jnp.int32, (B, RB), 1)
    eq = bb[None, :] == lax.broadcasted_iota(jnp.int32, (B, RB), 0)
    cand = jnp.where(eq, gid, -1)
    bmax = jnp.max(cand, axis=1)[None, :]

    @pl.when(i == 0)
    def _():
        last_ref[...] = jnp.full((1, B), -1, jnp.int32)

    last_ref[...] = jnp.maximum(last_ref[...], bmax)


def _gru_body(h_ref, a0_ref, a1_ref, last_ref, wggc_ref, wih_ref, whh_ref,
              bih_ref, bhh_ref, v_ref, sl_ref):
    i = pl.program_id(0)
    aggh = a0_ref[...] + a1_ref[...]
    agg = jnp.dot(aggh, wggc_ref[...], preferred_element_type=jnp.float32)
    h = h_ref[...]
    gi = jax.lax.dot_general(agg, wih_ref[...], (((1,), (1,)), ((), ())),
                             preferred_element_type=jnp.float32) + bih_ref[...]
    gh = jax.lax.dot_general(h, whh_ref[...], (((1,), (1,)), ((), ())),
                             preferred_element_type=jnp.float32) + bhh_ref[...]
    r = jax.nn.sigmoid(gi[:, :H] + gh[:, :H])
    z = jax.nn.sigmoid(gi[:, H:2 * H] + gh[:, H:2 * H])
    n = jnp.tanh(gi[:, 2 * H:] + r * gh[:, 2 * H:])
    v = (1.0 - z) * n + z * h
    v_ref[...] = v

    lastv = jnp.maximum(last_ref[0, :], 0)
    gid = i * RB + lax.broadcasted_iota(jnp.int32, (B, RB), 1)
    oh = (lastv[:, None] == gid).astype(jnp.float32)
    part = jnp.dot(oh, v, preferred_element_type=jnp.float32)

    @pl.when(i == 0)
    def _():
        sl_ref[...] = jnp.zeros_like(sl_ref)

    sl_ref[...] += part


def _sg_body(v_ref, b3_ref, sl_ref, w1_ref, w2_ref, b12_ref, q_ref, qb_ref,
             w3_ref, b3w_ref, o_ref, sh_ref):
    i = pl.program_id(0)
    bb = b3_ref[0, 0, :]
    v = v_ref[...]
    oh = (bb[:, None] == lax.broadcasted_iota(jnp.int32, (RB, B), 1)
          ).astype(jnp.float32)
    v_n = jnp.dot(oh, sl_ref[...], preferred_element_type=jnp.float32)
    pre = (jax.lax.dot_general(v_n, w1_ref[...], (((1,), (1,)), ((), ())),
                               preferred_element_type=jnp.float32)
           + jax.lax.dot_general(v, w2_ref[...], (((1,), (1,)), ((), ())),
                                 preferred_element_type=jnp.float32)
           + b12_ref[...])
    sig = jax.nn.sigmoid(pre)
    alpha = jnp.dot(sig, q_ref[...],
                    preferred_element_type=jnp.float32) + qb_ref[0, 0]
    contrib = alpha * v
    part = jax.lax.dot_general(oh, contrib, (((0,), (0,)), ((), ())),
                               preferred_element_type=jnp.float32)

    @pl.when(i == 0)
    def _():
        o_ref[...] = jnp.zeros_like(o_ref)

    o_ref[...] += part

    @pl.when(i == pl.num_programs(0) - 1)
    def _():
        sh_ref[...] = (
            jax.lax.dot_general(sl_ref[...], w3_ref[:, :H],
                                (((1,), (1,)), ((), ())),
                                preferred_element_type=jnp.float32)
            + jax.lax.dot_general(o_ref[...], w3_ref[:, H:],
                                  (((1,), (1,)), ((), ())),
                                  preferred_element_type=jnp.float32)
            + b3w_ref[...])


def _z_body(sh_ref, emb_ref, o_ref):
    o_ref[...] = jax.lax.dot_general(emb_ref[...], sh_ref[...],
                                     (((1,), (1,)), ((), ())),
                                     preferred_element_type=jnp.float32)


def kernel(x, edge_index, edge_weight, batch, embed, W_ggc, W_ih, W_hh, b_ih,
           b_hh, W1_w, W1_b, W2_w, W2_b, q_w, q_b, W3_w, W3_b):
    xp = jnp.concatenate([x, jnp.zeros((NP - N,), x.dtype)])
    x3 = xp.astype(jnp.int32).reshape(NW, 4, 80)
    pad_e = GCH * CK - E
    pad_idx = ((jnp.arange(pad_e, dtype=jnp.int32) * 97) % N)
    sd = jnp.concatenate(
        [edge_index.astype(jnp.int32), jnp.stack([pad_idx, pad_idx])],
        axis=1).reshape(2, GCH, CK)
    wdata = jnp.concatenate(
        [edge_weight, jnp.zeros((pad_e,), jnp.float32)])
    batch_p = jnp.concatenate(
        [batch.astype(jnp.int32), jnp.full((NP - N,), 2**30, jnp.int32)])
    batch3 = batch_p.reshape(NP // RB, 1, RB)
    bih2 = b_ih.reshape(1, 3 * H)
    bhh2 = b_hh.reshape(1, 3 * H)
    b12 = (W1_b + W2_b).reshape(1, H)
    qb2 = q_b.reshape(1, 1)
    qmat = jnp.broadcast_to(q_w.reshape(H, 1), (H, H))
    b3w = W3_b.reshape(1, H)

    h = _sc_gather(x3, embed)

    nb = NP // RB

    last = pl.pallas_call(
        _last_body,
        grid=(nb,),
        in_specs=[pl.BlockSpec((1, 1, RB), lambda i: (i, 0, 0))],
        out_specs=pl.BlockSpec((1, B), lambda i: (0, 0)),
        out_shape=jax.ShapeDtypeStruct((1, B), jnp.int32),
    )(batch3)

    agg2 = _sc_edges(sd, wdata, h)

    v, s_l = pl.pallas_call(
        _gru_body,
        grid=(nb,),
        in_specs=[
            pl.BlockSpec((RB, H), lambda i: (i, 0)),
            pl.BlockSpec((RB, H), lambda i: (i, 0)),
            pl.BlockSpec((RB, H), lambda i: (i + nb, 0)),
            pl.BlockSpec((1, B), lambda i: (0, 0)),
            pl.BlockSpec((H, H), lambda i: (0, 0)),
            pl.BlockSpec((3 * H, H), lambda i: (0, 0)),
            pl.BlockSpec((3 * H, H), lambda i: (0, 0)),
            pl.BlockSpec((1, 3 * H), lambda i: (0, 0)),
            pl.BlockSpec((1, 3 * H), lambda i: (0, 0)),
        ],
        out_specs=[pl.BlockSpec((RB, H), lambda i: (i, 0)),
                   pl.BlockSpec((B, H), lambda i: (0, 0))],
        out_shape=[jax.ShapeDtypeStruct((NP, H), jnp.float32),
                   jax.ShapeDtypeStruct((B, H), jnp.float32)],
    )(h, agg2, agg2, last, W_ggc, W_ih, W_hh, bih2, bhh2)

    s_g, s_h = pl.pallas_call(
        _sg_body,
        grid=(nb,),
        in_specs=[
            pl.BlockSpec((RB, H), lambda i: (i, 0)),
            pl.BlockSpec((1, 1, RB), lambda i: (i, 0, 0)),
            pl.BlockSpec((B, H), lambda i: (0, 0)),
            pl.BlockSpec((H, H), lambda i: (0, 0)),
            pl.BlockSpec((H, H), lambda i: (0, 0)),
            pl.BlockSpec((1, H), lambda i: (0, 0)),
            pl.BlockSpec((H, H), lambda i: (0, 0)),
            pl.BlockSpec(memory_space=pltpu.SMEM),
            pl.BlockSpec((H, 2 * H), lambda i: (0, 0)),
            pl.BlockSpec((1, H), lambda i: (0, 0)),
        ],
        out_specs=[pl.BlockSpec((B, H), lambda i: (0, 0)),
                   pl.BlockSpec((B, H), lambda i: (0, 0))],
        out_shape=[jax.ShapeDtypeStruct((B, H), jnp.float32),
                   jax.ShapeDtypeStruct((B, H), jnp.float32)],
    )(v, batch3, s_l, W1_w, W2_w, b12, qmat, qb2, W3_w, b3w)
    del s_g

    nvb = -(-NV // VB)
    zt = pl.pallas_call(
        _z_body,
        grid=(nvb,),
        in_specs=[pl.BlockSpec((B, H), lambda i: (0, 0)),
                  pl.BlockSpec((VB, H), lambda i: (i, 0))],
        out_specs=pl.BlockSpec((VB, B), lambda i: (i, 0)),
        out_shape=jax.ShapeDtypeStruct((NV, B), jnp.float32),
    )(s_h, embed)

    return zt.T
